# baseline (speedup 1.0000x reference)
import jax
import jax.numpy as jnp
from jax.experimental import pallas as pl
from jax.experimental.pallas import tpu as pltpu

_H = 256
_D = 1024
_T = 64
_C = 8
_G = 2
_MAX_ADAPTIVE_LR = 0.1

_NT = (((1,), (1,)), ((), ()))
_TN = (((0,), (0,)), ((), ()))


def _sig(z):
    return jax.nn.sigmoid(z)


def _dot(a, b, dims=None):
    if dims is None:
        return jnp.dot(a, b, preferred_element_type=jnp.float32)
    return jax.lax.dot_general(a, b, dims, preferred_element_type=jnp.float32)


def _wsum(w, rows):
    acc = w[:, 0:1] * rows[0]
    for j in range(1, len(rows)):
        acc += w[:, j:j + 1] * rows[j]
    return acc


def _nm_kernel(x_ref, wt_ref, w0_ref, p0_ref, w1t_ref, p1t_ref, out_ref,
               act_ref, s_ref, am_ref, bmf_ref, bmr_ref, bml_ref):
    lane = jax.lax.broadcasted_iota(jnp.int32, (1, 128), 1)
    sgate = jnp.where(lane == 1, _MAX_ADAPTIVE_LR, 1.0)
    for i in range(_G):
        xb = x_ref[i]
        fp = _dot(xb, wt_ref[...])
        q = fp[:, 0:_H]
        k = fp[:, _H:2 * _H]
        v = fp[:, 2 * _H:3 * _H]
        scal = fp[:, 3 * _H:3 * _H + 128]

        q = q * _sig(q)
        k = k * _sig(k)
        v = v * _sig(v)
        qn = jnp.sqrt(jnp.sum(q * q, axis=1, keepdims=True))
        kn = jnp.sqrt(jnp.sum(k * k, axis=1, keepdims=True))
        q = q / jnp.maximum(qn, 1e-12)
        k = k / jnp.maximum(kn, 1e-12)

        act_ref[i, :, 0:_H] = q
        act_ref[i, :, _H:2 * _H] = k
        act_ref[i, :, 2 * _H:3 * _H] = v
        s_ref[i] = _sig(scal) * sgate

        a_init = w0_ref[...] + p0_ref[...]
        b_init = w1t_ref[...] + p1t_ref[...]
        zmat = jnp.zeros((_H, _D), jnp.float32)
        b16 = b_init.astype(jnp.bfloat16)
        z16 = jnp.zeros((_H, _D), jnp.bfloat16)
        am_ref[i, :, 0:_D] = a_init
        am_ref[i, :, _D:2 * _D] = zmat
        bmf_ref[i, 0:_H, :] = b_init
        bmf_ref[i, _H:2 * _H, :] = zmat
        bmr_ref[i, 0:_H, :] = b16
        bmr_ref[i, _H:2 * _H, :] = z16
        bml_ref[i, :, 0:_D] = b16
        bml_ref[i, :, _D:2 * _D] = z16

    inv_h2 = 2.0 / _H
    one = jnp.ones((1, 1), jnp.float32)
    zero = jnp.zeros((1, 1), jnp.float32)
    zrow = jnp.zeros((1, _C), jnp.float32)
    zcol = jnp.zeros((_C, 1), jnp.float32)
    eye_r = [(jax.lax.broadcasted_iota(jnp.int32, (1, _C), 1) == s
              ).astype(jnp.float32) for s in range(_C)]
    eye_c = [(jax.lax.broadcasted_iota(jnp.int32, (_C, 1), 0) == s
              ).astype(jnp.float32) for s in range(_C)]

    def chunk(i, t0):
        kc = act_ref[i, pl.ds(t0, _C), _H:2 * _H]
        qc = act_ref[i, pl.ds(t0, _C), 0:_H]
        vc = act_ref[i, pl.ds(t0, _C), 2 * _H:3 * _H]
        sc = s_ref[i, pl.ds(t0, _C), :]
        kq = jnp.concatenate([kc, qc], axis=0)

        uu = _dot(kq, am_ref[i])
        u0, um = uu[0:_C, 0:_D], uu[0:_C, _D:2 * _D]
        q0, qm = uu[_C:2 * _C, 0:_D], uu[_C:2 * _C, _D:2 * _D]
        gg = _dot(kq, kc, _NT)
        kk, qk = gg[0:_C, :], gg[_C:2 * _C, :]

        lam, sigc, en = one, zero, one
        beta_r, m_r = zrow, zrow
        beta_c, m_c = zcol, zcol
        dpre_l, h_l, dp_l, h2_l = [], [], [], []
        lam_l, sig_l, brow_l = [], [], []

        for s in range(_C):
            v_r = vc[s:s + 1, :]
            a_t = sc[s:s + 1, 0:1]
            th_t = sc[s:s + 1, 1:2]
            e_t = sc[s:s + 1, 2:3]
            keep = 1.0 - a_t
            c_t = 2.0 * th_t

            u = lam * u0[s:s + 1, :] + sigc * um[s:s + 1, :]
            if s:
                u += _wsum(beta_r * kk[s:s + 1, :], dpre_l)
            su = _sig(u)
            h = u * su
            h16 = h.astype(jnp.bfloat16)

            pb = _dot(h16, bmr_ref[i], _NT)
            pred = lam * pb[:, 0:_H] + sigc * pb[:, _H:2 * _H]
            if s:
                hg = _dot(h, jnp.concatenate(h_l, axis=0), _NT)
                pred += _wsum(hg * beta_r[:, :s], dp_l)
            dpred = inv_h2 * (pred - v_r)

            db = _dot(dpred.astype(jnp.bfloat16), bml_ref[i])
            dh = lam * db[:, 0:_D] + sigc * db[:, _D:2 * _D]
            if s:
                dg = _dot(dpred, jnp.concatenate(dp_l, axis=0), _NT)
                dh += _wsum(dg * beta_r[:, :s], h_l)
            dpre = dh * (su * (1.0 + u * (1.0 - su)))

            en = e_t * en
            m_r = e_t * m_r - c_t * eye_r[s]
            m_c = e_t * m_c - c_t * eye_c[s]
            beta_r = keep * beta_r + m_r
            beta_c = keep * beta_c + m_c
            sigc = keep * sigc + en
            lam = keep * lam

            dpre_l.append(dpre)
            h_l.append(h)
            dp_l.append(dpred)

            u2 = (lam * q0[s:s + 1, :] + sigc * qm[s:s + 1, :]
                  + _wsum(beta_r * qk[s:s + 1, :], dpre_l))
            h2 = u2 * _sig(u2)
            h2_l.append(h2)
            lam_l.append(lam)
            sig_l.append(sigc)
            brow_l.append(beta_r)

        dpre_m = jnp.concatenate(dpre_l, axis=0)
        h_m = jnp.concatenate(h_l, axis=0)
        dp_m = jnp.concatenate(dp_l, axis=0)
        h2_m = jnp.concatenate(h2_l, axis=0)
        lam_col = jnp.concatenate(lam_l, axis=0).reshape(_C, 1)
        sig_col = jnp.concatenate(sig_l, axis=0).reshape(_C, 1)
        bmat = jnp.concatenate(brow_l, axis=0)

        ob = _dot(h2_m.astype(jnp.bfloat16), bmr_ref[i], _NT)
        outs = (lam_col * ob[:, 0:_H] + sig_col * ob[:, _H:2 * _H]
                + _dot(_dot(h2_m, h_m, _NT) * bmat, dp_m))
        out_ref[i, pl.ds(t0, _C), :, :] = outs.reshape(_C, 1, _H)

        d1 = _dot(jnp.concatenate([kc * beta_c, kc * m_c], axis=1),
                  dpre_m, _TN)
        a_old = am_ref[i, :, 0:_D]
        m_old = am_ref[i, :, _D:2 * _D]
        a_new = lam * a_old + sigc * m_old + d1[0:_H, :]
        m_new = en * m_old + d1[_H:2 * _H, :]
        am_ref[i, :, 0:_D] = a_new
        am_ref[i, :, _D:2 * _D] = m_new

        d2 = _dot(jnp.concatenate([dp_m * beta_c, dp_m * m_c], axis=1),
                  h_m, _TN)
        b_old = bmf_ref[i, 0:_H, :]
        mb_old = bmf_ref[i, _H:2 * _H, :]
        b_new = lam * b_old + sigc * mb_old + d2[0:_H, :]
        mb_new = en * mb_old + d2[_H:2 * _H, :]
        bmf_ref[i, 0:_H, :] = b_new
        bmf_ref[i, _H:2 * _H, :] = mb_new
        b_16 = b_new.astype(jnp.bfloat16)
        mb_16 = mb_new.astype(jnp.bfloat16)
        bmr_ref[i, 0:_H, :] = b_16
        bmr_ref[i, _H:2 * _H, :] = mb_16
        bml_ref[i, :, 0:_D] = b_16
        bml_ref[i, :, _D:2 * _D] = mb_16

    def step(c, carry):
        t0 = c * _C
        for i in range(_G):
            chunk(i, t0)
        return carry

    jax.lax.fori_loop(0, _T // _C, step, 0)


@jax.jit
def kernel(x, W_fused, W0, W1, P0, P1):
    B, T, H = x.shape
    D = W0.shape[1]

    wqkv_t = W_fused[:3 * H].T
    wscal_t = jnp.zeros((H, 128), W_fused.dtype).at[:, :3].set(
        W_fused[3 * H:3 * H + 3].T)
    wt = jnp.concatenate([wqkv_t, wscal_t], axis=1)

    grid = (B // _G,)
    out = pl.pallas_call(
        _nm_kernel,
        grid=grid,
        in_specs=[
            pl.BlockSpec((_G, T, H), lambda g: (g, 0, 0)),
            pl.BlockSpec((H, 3 * H + 128), lambda g: (0, 0)),
            pl.BlockSpec((H, D), lambda g: (0, 0)),
            pl.BlockSpec((H, D), lambda g: (0, 0)),
            pl.BlockSpec((H, D), lambda g: (0, 0)),
            pl.BlockSpec((H, D), lambda g: (0, 0)),
        ],
        out_specs=pl.BlockSpec((_G, T, 1, H), lambda g: (g, 0, 0, 0)),
        out_shape=jax.ShapeDtypeStruct((B, T, 1, H), jnp.float32),
        scratch_shapes=[
            pltpu.VMEM((_G, T, 3 * _H), jnp.float32),
            pltpu.VMEM((_G, T, 128), jnp.float32),
            pltpu.VMEM((_G, _H, 2 * _D), jnp.float32),
            pltpu.VMEM((_G, 2 * _H, _D), jnp.float32),
            pltpu.VMEM((_G, 2 * _H, _D), jnp.bfloat16),
            pltpu.VMEM((_G, _H, 2 * _D), jnp.bfloat16),
        ],
        compiler_params=pltpu.CompilerParams(
            dimension_semantics=("parallel",),
        ),
    )(x, wt, W0, P0, W1.T, P1.T)
    return out.reshape(B, T, H)


# speedup vs baseline: 1.0943x; 1.0943x over previous
import jax
import jax.numpy as jnp
from jax.experimental import pallas as pl
from jax.experimental.pallas import tpu as pltpu

_H = 256
_D = 1024
_T = 64
_C = 16
_G = 2
_MAX_ADAPTIVE_LR = 0.1

_NT = (((1,), (1,)), ((), ()))
_TN = (((0,), (0,)), ((), ()))


def _sig(z):
    return jax.nn.sigmoid(z)


def _dot(a, b, dims=None):
    if dims is None:
        return jnp.dot(a, b, preferred_element_type=jnp.float32)
    return jax.lax.dot_general(a, b, dims, preferred_element_type=jnp.float32)


def _wsum(w, rows):
    acc = w[:, 0:1] * rows[0]
    for j in range(1, len(rows)):
        acc += w[:, j:j + 1] * rows[j]
    return acc


def _nm_kernel(x_ref, wt_ref, w0_ref, p0_ref, w1t_ref, p1t_ref, out_ref,
               act_ref, s_ref, am_ref, bmr_ref, bml_ref):
    lane = jax.lax.broadcasted_iota(jnp.int32, (1, 128), 1)
    sgate = jnp.where(lane == 1, _MAX_ADAPTIVE_LR, 1.0)
    for i in range(_G):
        xb = x_ref[i]
        fp = _dot(xb, wt_ref[...])
        q = fp[:, 0:_H]
        k = fp[:, _H:2 * _H]
        v = fp[:, 2 * _H:3 * _H]
        scal = fp[:, 3 * _H:3 * _H + 128]

        q = q * _sig(q)
        k = k * _sig(k)
        v = v * _sig(v)
        qn = jnp.sqrt(jnp.sum(q * q, axis=1, keepdims=True))
        kn = jnp.sqrt(jnp.sum(k * k, axis=1, keepdims=True))
        q = q / jnp.maximum(qn, 1e-12)
        k = k / jnp.maximum(kn, 1e-12)

        act_ref[i, :, 0:_H] = q
        act_ref[i, :, _H:2 * _H] = k
        act_ref[i, :, 2 * _H:3 * _H] = v
        s_ref[i] = _sig(scal) * sgate

        a_init = w0_ref[...] + p0_ref[...]
        b_init = w1t_ref[...] + p1t_ref[...]
        zmat = jnp.zeros((_H, _D), jnp.float32)
        am_ref[i, :, 0:_D] = a_init
        am_ref[i, :, _D:2 * _D] = zmat
        bmr_ref[i, 0:_H, :] = b_init
        bmr_ref[i, _H:2 * _H, :] = zmat
        bml_ref[i, :, 0:_D] = b_init
        bml_ref[i, :, _D:2 * _D] = zmat

    inv_h2 = 2.0 / _H
    one = jnp.ones((1, 1), jnp.float32)
    zero = jnp.zeros((1, 1), jnp.float32)
    zrow = jnp.zeros((1, _C), jnp.float32)
    zcol = jnp.zeros((_C, 1), jnp.float32)
    eye_r = [(jax.lax.broadcasted_iota(jnp.int32, (1, _C), 1) == s
              ).astype(jnp.float32) for s in range(_C)]
    eye_c = [(jax.lax.broadcasted_iota(jnp.int32, (_C, 1), 0) == s
              ).astype(jnp.float32) for s in range(_C)]

    def chunk(i, t0):
        kc = act_ref[i, pl.ds(t0, _C), _H:2 * _H]
        qc = act_ref[i, pl.ds(t0, _C), 0:_H]
        vc = act_ref[i, pl.ds(t0, _C), 2 * _H:3 * _H]
        sc = s_ref[i, pl.ds(t0, _C), :]
        kq = jnp.concatenate([kc, qc], axis=0)

        uu = _dot(kq, am_ref[i])
        u0, um = uu[0:_C, 0:_D], uu[0:_C, _D:2 * _D]
        q0, qm = uu[_C:2 * _C, 0:_D], uu[_C:2 * _C, _D:2 * _D]
        gg = _dot(kq, kc, _NT)
        kk, qk = gg[0:_C, :], gg[_C:2 * _C, :]

        lam, sigc, en = one, zero, one
        beta_r, m_r = zrow, zrow
        beta_c, m_c = zcol, zcol
        dpre_l, h_l, dp_l, h2_l = [], [], [], []
        lam_l, sig_l, brow_l = [], [], []

        for s in range(_C):
            v_r = vc[s:s + 1, :]
            a_t = sc[s:s + 1, 0:1]
            th_t = sc[s:s + 1, 1:2]
            e_t = sc[s:s + 1, 2:3]
            keep = 1.0 - a_t
            c_t = 2.0 * th_t

            u = lam * u0[s:s + 1, :] + sigc * um[s:s + 1, :]
            if s:
                u += _wsum(beta_r * kk[s:s + 1, :], dpre_l)
            su = _sig(u)
            h = u * su

            pb = _dot(h, bmr_ref[i], _NT)
            pred = lam * pb[:, 0:_H] + sigc * pb[:, _H:2 * _H]
            if s:
                hg = _dot(h, jnp.concatenate(h_l, axis=0), _NT)
                pred += _wsum(hg * beta_r[:, :s], dp_l)
            dpred = inv_h2 * (pred - v_r)

            db = _dot(dpred, bml_ref[i])
            dh = lam * db[:, 0:_D] + sigc * db[:, _D:2 * _D]
            if s:
                dg = _dot(dpred, jnp.concatenate(dp_l, axis=0), _NT)
                dh += _wsum(dg * beta_r[:, :s], h_l)
            dpre = dh * (su * (1.0 + u * (1.0 - su)))

            en = e_t * en
            m_r = e_t * m_r - c_t * eye_r[s]
            m_c = e_t * m_c - c_t * eye_c[s]
            beta_r = keep * beta_r + m_r
            beta_c = keep * beta_c + m_c
            sigc = keep * sigc + en
            lam = keep * lam

            dpre_l.append(dpre)
            h_l.append(h)
            dp_l.append(dpred)

            u2 = (lam * q0[s:s + 1, :] + sigc * qm[s:s + 1, :]
                  + _wsum(beta_r * qk[s:s + 1, :], dpre_l))
            h2 = u2 * _sig(u2)
            h2_l.append(h2)
            lam_l.append(lam)
            sig_l.append(sigc)
            brow_l.append(beta_r)

        dpre_m = jnp.concatenate(dpre_l, axis=0)
        h_m = jnp.concatenate(h_l, axis=0)
        dp_m = jnp.concatenate(dp_l, axis=0)
        h2_m = jnp.concatenate(h2_l, axis=0)
        lam_col = jnp.concatenate(lam_l, axis=0).reshape(_C, 1)
        sig_col = jnp.concatenate(sig_l, axis=0).reshape(_C, 1)
        bmat = jnp.concatenate(brow_l, axis=0)

        ob = _dot(h2_m, bmr_ref[i], _NT)
        outs = (lam_col * ob[:, 0:_H] + sig_col * ob[:, _H:2 * _H]
                + _dot(_dot(h2_m, h_m, _NT) * bmat, dp_m))
        out_ref[i, pl.ds(t0, _C), :, :] = outs.reshape(_C, 1, _H)

        d1 = _dot(jnp.concatenate([kc * beta_c, kc * m_c], axis=1),
                  dpre_m, _TN)
        a_old = am_ref[i, :, 0:_D]
        m_old = am_ref[i, :, _D:2 * _D]
        a_new = lam * a_old + sigc * m_old + d1[0:_H, :]
        m_new = en * m_old + d1[_H:2 * _H, :]
        am_ref[i, :, 0:_D] = a_new
        am_ref[i, :, _D:2 * _D] = m_new

        d2 = _dot(jnp.concatenate([dp_m * beta_c, dp_m * m_c], axis=1),
                  h_m, _TN)
        b_old = bmr_ref[i, 0:_H, :]
        mb_old = bmr_ref[i, _H:2 * _H, :]
        b_new = lam * b_old + sigc * mb_old + d2[0:_H, :]
        mb_new = en * mb_old + d2[_H:2 * _H, :]
        bmr_ref[i, 0:_H, :] = b_new
        bmr_ref[i, _H:2 * _H, :] = mb_new
        bml_ref[i, :, 0:_D] = b_new
        bml_ref[i, :, _D:2 * _D] = mb_new

    def step(c, carry):
        t0 = c * _C
        for i in range(_G):
            chunk(i, t0)
        return carry

    jax.lax.fori_loop(0, _T // _C, step, 0)


@jax.jit
def kernel(x, W_fused, W0, W1, P0, P1):
    B, T, H = x.shape
    D = W0.shape[1]

    wqkv_t = W_fused[:3 * H].T
    wscal_t = jnp.zeros((H, 128), W_fused.dtype).at[:, :3].set(
        W_fused[3 * H:3 * H + 3].T)
    wt = jnp.concatenate([wqkv_t, wscal_t], axis=1)

    grid = (B // _G,)
    out = pl.pallas_call(
        _nm_kernel,
        grid=grid,
        in_specs=[
            pl.BlockSpec((_G, T, H), lambda g: (g, 0, 0)),
            pl.BlockSpec((H, 3 * H + 128), lambda g: (0, 0)),
            pl.BlockSpec((H, D), lambda g: (0, 0)),
            pl.BlockSpec((H, D), lambda g: (0, 0)),
            pl.BlockSpec((H, D), lambda g: (0, 0)),
            pl.BlockSpec((H, D), lambda g: (0, 0)),
        ],
        out_specs=pl.BlockSpec((_G, T, 1, H), lambda g: (g, 0, 0, 0)),
        out_shape=jax.ShapeDtypeStruct((B, T, 1, H), jnp.float32),
        scratch_shapes=[
            pltpu.VMEM((_G, T, 3 * _H), jnp.float32),
            pltpu.VMEM((_G, T, 128), jnp.float32),
            pltpu.VMEM((_G, _H, 2 * _D), jnp.float32),
            pltpu.VMEM((_G, 2 * _H, _D), jnp.float32),
            pltpu.VMEM((_G, _H, 2 * _D), jnp.float32),
        ],
        compiler_params=pltpu.CompilerParams(
            dimension_semantics=("parallel",),
        ),
    )(x, wt, W0, P0, W1.T, P1.T)
    return out.reshape(B, T, H)


# speedup vs baseline: 1.1743x; 1.0732x over previous
import jax
import jax.numpy as jnp
from jax.experimental import pallas as pl
from jax.experimental.pallas import tpu as pltpu

_H = 256
_D = 1024
_T = 64
_C = 16
_G = 2
_MAX_ADAPTIVE_LR = 0.1

_NT = (((1,), (1,)), ((), ()))
_TN = (((0,), (0,)), ((), ()))


def _sig(z):
    return jax.nn.sigmoid(z)


def _dot(a, b, dims=None):
    if dims is None:
        return jnp.dot(a, b, preferred_element_type=jnp.float32)
    return jax.lax.dot_general(a, b, dims, preferred_element_type=jnp.float32)


def _wsum(w, rows):
    acc = w[:, 0:1] * rows[0]
    for j in range(1, len(rows)):
        acc += w[:, j:j + 1] * rows[j]
    return acc


def _nm_kernel(x_ref, wt_ref, w0_ref, p0_ref, w1t_ref, p1t_ref,
               w1_ref, p1_ref, out_ref,
               act_ref, s_ref, am_ref, btl_ref, bml_ref):
    lane = jax.lax.broadcasted_iota(jnp.int32, (1, 128), 1)
    sgate = jnp.where(lane == 1, _MAX_ADAPTIVE_LR, 1.0)
    for i in range(_G):
        xb = x_ref[i]
        fp = _dot(xb, wt_ref[...])
        q = fp[:, 0:_H]
        k = fp[:, _H:2 * _H]
        v = fp[:, 2 * _H:3 * _H]
        scal = fp[:, 3 * _H:3 * _H + 128]

        q = q * _sig(q)
        k = k * _sig(k)
        v = v * _sig(v)
        qn = jnp.sqrt(jnp.sum(q * q, axis=1, keepdims=True))
        kn = jnp.sqrt(jnp.sum(k * k, axis=1, keepdims=True))
        q = q / jnp.maximum(qn, 1e-12)
        k = k / jnp.maximum(kn, 1e-12)

        act_ref[i, :, 0:_H] = q
        act_ref[i, :, _H:2 * _H] = k
        act_ref[i, :, 2 * _H:3 * _H] = v
        s_ref[i] = _sig(scal) * sgate

        a_init = w0_ref[...] + p0_ref[...]
        b_init = w1t_ref[...] + p1t_ref[...]
        bt_init = w1_ref[...] + p1_ref[...]
        zmat = jnp.zeros((_H, _D), jnp.float32)
        am_ref[i, :, 0:_D] = a_init
        am_ref[i, :, _D:2 * _D] = zmat
        btl_ref[i, :, 0:_H] = bt_init
        btl_ref[i, :, _H:2 * _H] = jnp.zeros((_D, _H), jnp.float32)
        bml_ref[i, :, 0:_D] = b_init
        bml_ref[i, :, _D:2 * _D] = zmat

    inv_h2 = 2.0 / _H
    one = jnp.ones((1, 1), jnp.float32)
    zero = jnp.zeros((1, 1), jnp.float32)
    zrow = jnp.zeros((1, _C), jnp.float32)
    zcol = jnp.zeros((_C, 1), jnp.float32)
    eye_r = [(jax.lax.broadcasted_iota(jnp.int32, (1, _C), 1) == s
              ).astype(jnp.float32) for s in range(_C)]
    eye_c = [(jax.lax.broadcasted_iota(jnp.int32, (_C, 1), 0) == s
              ).astype(jnp.float32) for s in range(_C)]

    def chunk(i, t0):
        kc = act_ref[i, pl.ds(t0, _C), _H:2 * _H]
        qc = act_ref[i, pl.ds(t0, _C), 0:_H]
        vc = act_ref[i, pl.ds(t0, _C), 2 * _H:3 * _H]
        sc = s_ref[i, pl.ds(t0, _C), :]
        kq = jnp.concatenate([kc, qc], axis=0)

        uu = _dot(kq, am_ref[i])
        u0, um = uu[0:_C, 0:_D], uu[0:_C, _D:2 * _D]
        q0, qm = uu[_C:2 * _C, 0:_D], uu[_C:2 * _C, _D:2 * _D]
        gg = _dot(kq, kc, _NT)
        kk, qk = gg[0:_C, :], gg[_C:2 * _C, :]

        lam, sigc, en = one, zero, one
        beta_r, m_r = zrow, zrow
        beta_c, m_c = zcol, zcol
        dpre_l, h_l, dp_l, h2_l = [], [], [], []
        lam_l, sig_l, brow_l = [], [], []

        for s in range(_C):
            v_r = vc[s:s + 1, :]
            a_t = sc[s:s + 1, 0:1]
            th_t = sc[s:s + 1, 1:2]
            e_t = sc[s:s + 1, 2:3]
            keep = 1.0 - a_t
            c_t = 2.0 * th_t

            u = lam * u0[s:s + 1, :] + sigc * um[s:s + 1, :]
            if s:
                u += _wsum(beta_r * kk[s:s + 1, :], dpre_l)
            su = _sig(u)
            h = u * su

            pb = _dot(h, btl_ref[i])
            pred = lam * pb[:, 0:_H] + sigc * pb[:, _H:2 * _H]
            if s:
                hg = _dot(h, jnp.concatenate(h_l, axis=0), _NT)
                pred += _wsum(hg * beta_r[:, :s], dp_l)
            dpred = inv_h2 * (pred - v_r)

            db = _dot(dpred, bml_ref[i])
            dh = lam * db[:, 0:_D] + sigc * db[:, _D:2 * _D]
            if s:
                dg = _dot(dpred, jnp.concatenate(dp_l, axis=0), _NT)
                dh += _wsum(dg * beta_r[:, :s], h_l)
            dpre = dh * (su * (1.0 + u * (1.0 - su)))

            en = e_t * en
            m_r = e_t * m_r - c_t * eye_r[s]
            m_c = e_t * m_c - c_t * eye_c[s]
            beta_r = keep * beta_r + m_r
            beta_c = keep * beta_c + m_c
            sigc = keep * sigc + en
            lam = keep * lam

            dpre_l.append(dpre)
            h_l.append(h)
            dp_l.append(dpred)

            u2 = (lam * q0[s:s + 1, :] + sigc * qm[s:s + 1, :]
                  + _wsum(beta_r * qk[s:s + 1, :], dpre_l))
            h2 = u2 * _sig(u2)
            h2_l.append(h2)
            lam_l.append(lam)
            sig_l.append(sigc)
            brow_l.append(beta_r)

        dpre_m = jnp.concatenate(dpre_l, axis=0)
        h_m = jnp.concatenate(h_l, axis=0)
        dp_m = jnp.concatenate(dp_l, axis=0)
        h2_m = jnp.concatenate(h2_l, axis=0)
        lam_col = jnp.concatenate(lam_l, axis=0).reshape(_C, 1)
        sig_col = jnp.concatenate(sig_l, axis=0).reshape(_C, 1)
        bmat = jnp.concatenate(brow_l, axis=0)

        ob = _dot(h2_m, btl_ref[i])
        outs = (lam_col * ob[:, 0:_H] + sig_col * ob[:, _H:2 * _H]
                + _dot(_dot(h2_m, h_m, _NT) * bmat, dp_m))
        out_ref[i, pl.ds(t0, _C), :, :] = outs.reshape(_C, 1, _H)

        d1 = _dot(jnp.concatenate([kc * beta_c, kc * m_c], axis=1),
                  dpre_m, _TN)
        a_old = am_ref[i, :, 0:_D]
        m_old = am_ref[i, :, _D:2 * _D]
        a_new = lam * a_old + sigc * m_old + d1[0:_H, :]
        m_new = en * m_old + d1[_H:2 * _H, :]
        am_ref[i, :, 0:_D] = a_new
        am_ref[i, :, _D:2 * _D] = m_new

        d2 = _dot(jnp.concatenate([dp_m * beta_c, dp_m * m_c], axis=1),
                  h_m, _TN)
        b_old = bml_ref[i, :, 0:_D]
        mb_old = bml_ref[i, :, _D:2 * _D]
        b_new = lam * b_old + sigc * mb_old + d2[0:_H, :]
        mb_new = en * mb_old + d2[_H:2 * _H, :]
        bml_ref[i, :, 0:_D] = b_new
        bml_ref[i, :, _D:2 * _D] = mb_new

        d2t = _dot(jnp.concatenate([h_m * beta_c, h_m * m_c], axis=1),
                   dp_m, _TN)
        bt_old = btl_ref[i, :, 0:_H]
        mbt_old = btl_ref[i, :, _H:2 * _H]
        bt_new = lam * bt_old + sigc * mbt_old + d2t[0:_D, :]
        mbt_new = en * mbt_old + d2t[_D:2 * _D, :]
        btl_ref[i, :, 0:_H] = bt_new
        btl_ref[i, :, _H:2 * _H] = mbt_new

    def step(c, carry):
        t0 = c * _C
        for i in range(_G):
            chunk(i, t0)
        return carry

    jax.lax.fori_loop(0, _T // _C, step, 0)


@jax.jit
def kernel(x, W_fused, W0, W1, P0, P1):
    B, T, H = x.shape
    D = W0.shape[1]

    wqkv_t = W_fused[:3 * H].T
    wscal_t = jnp.zeros((H, 128), W_fused.dtype).at[:, :3].set(
        W_fused[3 * H:3 * H + 3].T)
    wt = jnp.concatenate([wqkv_t, wscal_t], axis=1)

    grid = (B // _G,)
    out = pl.pallas_call(
        _nm_kernel,
        grid=grid,
        in_specs=[
            pl.BlockSpec((_G, T, H), lambda g: (g, 0, 0)),
            pl.BlockSpec((H, 3 * H + 128), lambda g: (0, 0)),
            pl.BlockSpec((H, D), lambda g: (0, 0)),
            pl.BlockSpec((H, D), lambda g: (0, 0)),
            pl.BlockSpec((H, D), lambda g: (0, 0)),
            pl.BlockSpec((H, D), lambda g: (0, 0)),
            pl.BlockSpec((D, H), lambda g: (0, 0)),
            pl.BlockSpec((D, H), lambda g: (0, 0)),
        ],
        out_specs=pl.BlockSpec((_G, T, 1, H), lambda g: (g, 0, 0, 0)),
        out_shape=jax.ShapeDtypeStruct((B, T, 1, H), jnp.float32),
        scratch_shapes=[
            pltpu.VMEM((_G, T, 3 * _H), jnp.float32),
            pltpu.VMEM((_G, T, 128), jnp.float32),
            pltpu.VMEM((_G, _H, 2 * _D), jnp.float32),
            pltpu.VMEM((_G, _D, 2 * _H), jnp.float32),
            pltpu.VMEM((_G, _H, 2 * _D), jnp.float32),
        ],
        compiler_params=pltpu.CompilerParams(
            dimension_semantics=("parallel",),
        ),
    )(x, wt, W0, P0, W1.T, P1.T, W1, P1)
    return out.reshape(B, T, H)
